# Initial kernel scaffold; baseline (speedup 1.0000x reference)
import jax
import jax.numpy as jnp
from jax.experimental import pallas as pl


def kernel(queries, keys):
    raise NotImplementedError("write your pallas kernel here")


# speedup vs baseline: 2.8402x; 2.8402x over previous
import jax
import jax.numpy as jnp
from jax.experimental import pallas as pl

D = 75
DP = 128
KNN = 14
QBLK = 128


def _lidia_tc_body(q_ref, kT_ref, k_ref, agg_ref, ind_ref):
    f32 = jnp.float32

    q = (q_ref[...] - 0.5) * 2.0
    lane_q = jax.lax.broadcasted_iota(jnp.int32, (QBLK, DP), 1)
    qmean = jnp.sum(q, axis=1, keepdims=True) / float(D)
    qn = jnp.where(lane_q < D, q - qmean, 0.0)
    q2 = jnp.sum(qn * qn, axis=1, keepdims=True)

    kT = (kT_ref[...] - 0.5) * 2.0
    KTOT = kT.shape[1]
    sub = jax.lax.broadcasted_iota(jnp.int32, (DP, KTOT), 0)
    kmeanT = jnp.sum(kT, axis=0, keepdims=True) / float(D)
    kTn = jnp.where(sub < D, kT - kmeanT, 0.0)
    k2 = jnp.sum(kTn * kTn, axis=0, keepdims=True)

    t = jax.lax.dot_general(qn, kTn, (((1,), (0,)), ((), ())),
                            preferred_element_type=f32)
    nd = -((q2 + k2) - 2.0 * t)

    lane_wide = jax.lax.broadcasted_iota(jnp.int32, (QBLK, KTOT), 1)
    lane_out = jax.lax.broadcasted_iota(jnp.int32, (QBLK, DP), 1)

    W = jnp.zeros((QBLK, KTOT), f32)
    inds = jnp.zeros((QBLK, DP), jnp.int32)
    wsum = jnp.zeros((QBLK, 1), f32)
    m0 = None
    for j in range(KNN):
        m = jnp.max(nd, axis=1, keepdims=True)
        idx = jnp.argmax(nd, axis=1).reshape(QBLK, 1).astype(jnp.int32)
        onehot = lane_wide == idx
        if j == 0:
            m0 = m
            e = jnp.ones((QBLK, 1), f32)
        else:
            e = jnp.exp(m - m0)
        W = jnp.where(onehot, e, W)
        wsum = wsum + e
        inds = jnp.where(lane_out == j, idx, inds)
        nd = jnp.where(onehot, -jnp.inf, nd)

    k = (k_ref[...] - 0.5) * 2.0
    lane_k = jax.lax.broadcasted_iota(jnp.int32, (KTOT, DP), 1)
    kmean = jnp.sum(k, axis=1, keepdims=True) / float(D)
    kn = jnp.where(lane_k < D, k - kmean, 0.0)

    agg = jax.lax.dot_general(W, kn, (((1,), (0,)), ((), ())),
                              preferred_element_type=f32) / wsum
    agg_ref[...] = agg
    ind_ref[...] = inds


def _run_tc(qp, kTp, kp, interpret=False):
    Q = qp.shape[0]
    K = kp.shape[0]
    return pl.pallas_call(
        _lidia_tc_body,
        grid=(Q // QBLK,),
        in_specs=[
            pl.BlockSpec((QBLK, DP), lambda i: (i, 0)),
            pl.BlockSpec((DP, K), lambda i: (0, 0)),
            pl.BlockSpec((K, DP), lambda i: (0, 0)),
        ],
        out_specs=[
            pl.BlockSpec((QBLK, DP), lambda i: (i, 0)),
            pl.BlockSpec((QBLK, DP), lambda i: (i, 0)),
        ],
        out_shape=[
            jax.ShapeDtypeStruct((Q, DP), jnp.float32),
            jax.ShapeDtypeStruct((Q, DP), jnp.int32),
        ],
        interpret=interpret,
    )(qp, kTp, kp)


def kernel(queries, keys, interpret=False):
    qp = jnp.pad(queries, ((0, 0), (0, DP - D)), constant_values=0.5)
    kp = jnp.pad(keys, ((0, 0), (0, DP - D)), constant_values=0.5)
    kTp = jnp.pad(keys.T, ((0, DP - D), (0, 0)), constant_values=0.5)
    agg, inds = _run_tc(qp, kTp, kp, interpret=interpret)
    return agg[:, :D], inds[:, :KNN]


# speedup vs baseline: 4.1799x; 1.4717x over previous
import functools

import jax
import jax.numpy as jnp
from jax.experimental import pallas as pl
from jax.experimental.pallas import tpu as pltpu
from jax.experimental.pallas import tpu_sc as plsc

D = 75
DP = 128
KNN = 14
QBLK = 128


def _prep_body(k_ref, kT_ref, kn_ref, kTn_ref):
    KTOT = k_ref.shape[0]

    k = (k_ref[...] - 0.5) * 2.0
    lane = jax.lax.broadcasted_iota(jnp.int32, (KTOT, DP), 1)
    kmean = jnp.sum(k, axis=1, keepdims=True) / float(D)
    kn_ref[...] = jnp.where(lane < D, k - kmean, 0.0)

    kT = (kT_ref[...] - 0.5) * 2.0
    sub = jax.lax.broadcasted_iota(jnp.int32, (DP, KTOT), 0)
    kmeanT = jnp.sum(kT, axis=0, keepdims=True) / float(D)
    kTn = jnp.where(sub < D, kT - kmeanT, 0.0)
    k2 = jnp.sum(kTn * kTn, axis=0, keepdims=True)
    kTn_ref[...] = jnp.where(sub == DP - 1, k2, kTn)


def _run_prep(kp, kTp):
    K = kp.shape[0]
    return pl.pallas_call(
        _prep_body,
        in_specs=[
            pl.BlockSpec((K, DP), lambda: (0, 0)),
            pl.BlockSpec((DP, K), lambda: (0, 0)),
        ],
        out_specs=[
            pl.BlockSpec((K, DP), lambda: (0, 0)),
            pl.BlockSpec((DP, K), lambda: (0, 0)),
        ],
        out_shape=[
            jax.ShapeDtypeStruct((K, DP), jnp.float32),
            jax.ShapeDtypeStruct((DP, K), jnp.float32),
        ],
    )(kp, kTp)


def _main_body(q_ref, kTn_ref, ind_ref, w_ref):
    f32 = jnp.float32

    q = (q_ref[...] - 0.5) * 2.0
    lane_q = jax.lax.broadcasted_iota(jnp.int32, (QBLK, DP), 1)
    qmean = jnp.sum(q, axis=1, keepdims=True) / float(D)
    qn = jnp.where(lane_q < D, q - qmean, 0.0)
    q2 = jnp.sum(qn * qn, axis=1, keepdims=True)

    kTn = kTn_ref[...]
    KTOT = kTn.shape[1]
    k2 = kTn[DP - 1 : DP, :]

    t = jax.lax.dot_general(qn, kTn, (((1,), (0,)), ((), ())),
                            preferred_element_type=f32)
    nd = -((q2 + k2) - 2.0 * t)

    lane_wide = jax.lax.broadcasted_iota(jnp.int32, (QBLK, KTOT), 1)
    lane_out = jax.lax.broadcasted_iota(jnp.int32, (QBLK, DP), 1)

    inds = jnp.zeros((QBLK, DP), jnp.int32)
    w = jnp.zeros((QBLK, DP), f32)
    wsum = jnp.zeros((QBLK, 1), f32)
    m0 = None
    for j in range(KNN):
        m = jnp.max(nd, axis=1, keepdims=True)
        idx = jnp.min(jnp.where(nd == m, lane_wide, KTOT),
                      axis=1, keepdims=True)
        if j == 0:
            m0 = m
            e = jnp.ones((QBLK, 1), f32)
        else:
            e = jnp.exp(m - m0)
        inds = jnp.where(lane_out == j, idx, inds)
        w = jnp.where(lane_out == j, e, w)
        wsum = wsum + e
        nd = jnp.where(lane_wide == idx, -jnp.inf, nd)

    ind_ref[...] = inds
    w_ref[...] = w / wsum


def _run_main(qp, kTn):
    Q = qp.shape[0]
    K = kTn.shape[1]
    return pl.pallas_call(
        _main_body,
        grid=(Q // QBLK,),
        in_specs=[
            pl.BlockSpec((QBLK, DP), lambda i: (i, 0)),
            pl.BlockSpec((DP, K), lambda i: (0, 0)),
        ],
        out_specs=[
            pl.BlockSpec((QBLK, DP), lambda i: (i, 0)),
            pl.BlockSpec((QBLK, DP), lambda i: (i, 0)),
        ],
        out_shape=[
            jax.ShapeDtypeStruct((Q, DP), jnp.int32),
            jax.ShapeDtypeStruct((Q, DP), jnp.float32),
        ],
    )(qp, kTn)


GWIN = 128


def _run_sc_gather(kn, idx_flat):
    B = idx_flat.shape[0]
    idx2 = idx_flat.reshape(1, B)
    mesh = plsc.VectorSubcoreMesh(core_axis_name="c", subcore_axis_name="s")

    @functools.partial(
        pl.kernel,
        out_type=jax.ShapeDtypeStruct((B, DP), jnp.float32),
        mesh=mesh,
    )
    def _sc_kernel(kn_hbm, idx_hbm, out_hbm):
        def body(i_vmem, o_vmem):
            pltpu.sync_copy(kn_hbm.at[i_vmem.at[0]], o_vmem)

        pltpu.emit_pipeline(
            body,
            grid=(B // GWIN,),
            in_specs=[pl.BlockSpec((1, GWIN), index_map=lambda i: (0, i))],
            out_specs=[pl.BlockSpec((GWIN, DP), index_map=lambda i: (i, 0))],
            core_axis_name=("c", "s"),
            dimension_semantics=(pltpu.PARALLEL,),
        )(idx_hbm, out_hbm)

    return _sc_kernel(kn, idx2)


def _reduce_body(nbr_ref, w_ref, agg_ref):
    acc = nbr_ref[0] * w_ref[:, 0:1]
    for j in range(1, KNN):
        acc = acc + nbr_ref[j] * w_ref[:, j : j + 1]
    agg_ref[...] = acc


def _run_reduce(nbr_slabs, w):
    Q = w.shape[0]
    return pl.pallas_call(
        _reduce_body,
        grid=(Q // QBLK,),
        in_specs=[
            pl.BlockSpec((KNN, QBLK, DP), lambda i: (0, i, 0)),
            pl.BlockSpec((QBLK, DP), lambda i: (i, 0)),
        ],
        out_specs=pl.BlockSpec((QBLK, DP), lambda i: (i, 0)),
        out_shape=jax.ShapeDtypeStruct((Q, DP), jnp.float32),
    )(nbr_slabs, w)


def kernel(queries, keys):
    Q = queries.shape[0]
    K = keys.shape[0]
    qp = jnp.pad(queries, ((0, 0), (0, DP - D)), constant_values=0.5)
    kp = jnp.pad(keys, ((0, 0), (0, DP - D)), constant_values=0.5)
    kTp = jnp.pad(keys.T, ((0, DP - D), (0, 0)), constant_values=0.5)

    kn, kTn = _run_prep(kp, kTp)
    inds_pad, w_pad = _run_main(qp, kTn)
    inds = inds_pad[:, :KNN]

    idx_flat = inds.T.reshape(KNN * Q)
    nbr = _run_sc_gather(kn, idx_flat)
    nbr_slabs = nbr.reshape(KNN, Q, DP)

    agg = _run_reduce(nbr_slabs, w_pad)
    return agg[:, :D], inds


# speedup vs baseline: 4.4349x; 1.0610x over previous
import functools

import jax
import jax.numpy as jnp
from jax.experimental import pallas as pl
from jax.experimental.pallas import tpu as pltpu
from jax.experimental.pallas import tpu_sc as plsc

D = 75
DP = 128
KNN = 14
QBLK = 128


def _prep_body(k_ref, kT_ref, kn_ref, kTn_ref):
    KTOT = k_ref.shape[0]

    k = (k_ref[...] - 0.5) * 2.0
    lane = jax.lax.broadcasted_iota(jnp.int32, (KTOT, DP), 1)
    kmean = jnp.sum(k, axis=1, keepdims=True) / float(D)
    kn_ref[...] = jnp.where(lane < D, k - kmean, 0.0)

    kT = (kT_ref[...] - 0.5) * 2.0
    sub = jax.lax.broadcasted_iota(jnp.int32, (DP, KTOT), 0)
    kmeanT = jnp.sum(kT, axis=0, keepdims=True) / float(D)
    kTn = jnp.where(sub < D, kT - kmeanT, 0.0)
    k2 = jnp.sum(kTn * kTn, axis=0, keepdims=True)
    kTn_ref[...] = jnp.where(sub == DP - 1, k2, kTn)


def _run_prep(kp, kTp):
    K = kp.shape[0]
    return pl.pallas_call(
        _prep_body,
        in_specs=[
            pl.BlockSpec((K, DP), lambda: (0, 0)),
            pl.BlockSpec((DP, K), lambda: (0, 0)),
        ],
        out_specs=[
            pl.BlockSpec((K, DP), lambda: (0, 0)),
            pl.BlockSpec((DP, K), lambda: (0, 0)),
        ],
        out_shape=[
            jax.ShapeDtypeStruct((K, DP), jnp.float32),
            jax.ShapeDtypeStruct((DP, K), jnp.float32),
        ],
    )(kp, kTp)


def _main_body(q_ref, kTn_ref, ind_ref, w_ref):
    f32 = jnp.float32

    q = (q_ref[...] - 0.5) * 2.0
    lane_q = jax.lax.broadcasted_iota(jnp.int32, (QBLK, DP), 1)
    qmean = jnp.sum(q, axis=1, keepdims=True) / float(D)
    qn = jnp.where(lane_q < D, q - qmean, 0.0)
    q2 = jnp.sum(qn * qn, axis=1, keepdims=True)

    kTn = kTn_ref[...]
    KTOT = kTn.shape[1]
    k2 = kTn[DP - 1 : DP, :]

    t = jax.lax.dot_general(qn, kTn, (((1,), (0,)), ((), ())),
                            preferred_element_type=f32)
    nd = -((q2 + k2) - 2.0 * t)

    lane_out = jax.lax.broadcasted_iota(jnp.int32, (QBLK, DP), 1)

    HK = KTOT // 2
    lane_half = jax.lax.broadcasted_iota(jnp.int32, (QBLK, HK), 1)
    a = nd[:, :HK]
    b = nd[:, HK:]
    ge = a >= b
    V = jnp.where(ge, a, b)
    I = jnp.where(ge, lane_half, lane_half + HK)
    L = jnp.where(ge, b, a)
    J = jnp.where(ge, lane_half + HK, lane_half)

    inds = jnp.zeros((QBLK, DP), jnp.int32)
    w = jnp.zeros((QBLK, DP), f32)
    wsum = jnp.zeros((QBLK, 1), f32)
    m0 = None
    for j in range(KNN):
        m = jnp.max(V, axis=1, keepdims=True)
        idx = jnp.min(jnp.where(V == m, I, KTOT),
                      axis=1, keepdims=True)
        if j == 0:
            m0 = m
            e = jnp.ones((QBLK, 1), f32)
        else:
            e = jnp.exp(m - m0)
        inds = jnp.where(lane_out == j, idx, inds)
        w = jnp.where(lane_out == j, e, w)
        wsum = wsum + e
        hit = I == idx
        V = jnp.where(hit, L, V)
        I = jnp.where(hit, J, I)
        L = jnp.where(hit, -jnp.inf, L)

    ind_ref[...] = inds
    w_ref[...] = w / wsum


def _run_main(qp, kTn):
    Q = qp.shape[0]
    K = kTn.shape[1]
    return pl.pallas_call(
        _main_body,
        grid=(Q // QBLK,),
        in_specs=[
            pl.BlockSpec((QBLK, DP), lambda i: (i, 0)),
            pl.BlockSpec((DP, K), lambda i: (0, 0)),
        ],
        out_specs=[
            pl.BlockSpec((QBLK, DP), lambda i: (i, 0)),
            pl.BlockSpec((QBLK, DP), lambda i: (i, 0)),
        ],
        out_shape=[
            jax.ShapeDtypeStruct((Q, DP), jnp.int32),
            jax.ShapeDtypeStruct((Q, DP), jnp.float32),
        ],
    )(qp, kTn)


GWIN = 128


def _run_sc_gather(kn, idx_flat):
    B = idx_flat.shape[0]
    idx2 = idx_flat.reshape(1, B)
    mesh = plsc.VectorSubcoreMesh(core_axis_name="c", subcore_axis_name="s")

    @functools.partial(
        pl.kernel,
        out_type=jax.ShapeDtypeStruct((B, DP), jnp.float32),
        mesh=mesh,
    )
    def _sc_kernel(kn_hbm, idx_hbm, out_hbm):
        def body(i_vmem, o_vmem):
            pltpu.sync_copy(kn_hbm.at[i_vmem.at[0]], o_vmem)

        pltpu.emit_pipeline(
            body,
            grid=(B // GWIN,),
            in_specs=[pl.BlockSpec((1, GWIN), index_map=lambda i: (0, i))],
            out_specs=[pl.BlockSpec((GWIN, DP), index_map=lambda i: (i, 0))],
            core_axis_name=("c", "s"),
            dimension_semantics=(pltpu.PARALLEL,),
        )(idx_hbm, out_hbm)

    return _sc_kernel(kn, idx2)


def _reduce_body(nbr_ref, w_ref, agg_ref):
    acc = nbr_ref[0] * w_ref[:, 0:1]
    for j in range(1, KNN):
        acc = acc + nbr_ref[j] * w_ref[:, j : j + 1]
    agg_ref[...] = acc


def _run_reduce(nbr_slabs, w):
    Q = w.shape[0]
    return pl.pallas_call(
        _reduce_body,
        grid=(Q // QBLK,),
        in_specs=[
            pl.BlockSpec((KNN, QBLK, DP), lambda i: (0, i, 0)),
            pl.BlockSpec((QBLK, DP), lambda i: (i, 0)),
        ],
        out_specs=pl.BlockSpec((QBLK, DP), lambda i: (i, 0)),
        out_shape=jax.ShapeDtypeStruct((Q, DP), jnp.float32),
    )(nbr_slabs, w)


def kernel(queries, keys):
    Q = queries.shape[0]
    K = keys.shape[0]
    qp = jnp.pad(queries, ((0, 0), (0, DP - D)), constant_values=0.5)
    kp = jnp.pad(keys, ((0, 0), (0, DP - D)), constant_values=0.5)
    kTp = jnp.pad(keys.T, ((0, DP - D), (0, 0)), constant_values=0.5)

    kn, kTn = _run_prep(kp, kTp)
    inds_pad, w_pad = _run_main(qp, kTn)
    inds = inds_pad[:, :KNN]

    idx_flat = inds.T.reshape(KNN * Q)
    nbr = _run_sc_gather(kn, idx_flat)
    nbr_slabs = nbr.reshape(KNN, Q, DP)

    agg = _run_reduce(nbr_slabs, w_pad)
    return agg[:, :D], inds
